# baseline (speedup 1.0000x reference)
import functools

import jax
import jax.numpy as jnp
from jax import lax
from jax.experimental import pallas as pl
from jax.experimental.pallas import tpu as pltpu
from jax.experimental.pallas import tpu_sc as plsc

_N = 16384
_C = 1000
_BLK = 1024
_NW = 32
_YPW = _N // _NW


def _sc_hist_body(y_hbm, hist_hbm, yh_v, ones_v, zeros_v, hist_sh):
    cid = lax.axis_index("c")
    sid = lax.axis_index("s")
    wid = sid * 2 + cid

    ones16i = jnp.ones((16,), jnp.int32)
    pltpu.sync_copy(y_hbm.at[pl.ds(wid * _YPW, _YPW)], yh_v)

    def _init(i, _):
        ones_v[pl.ds(i * 16, 16)] = ones16i
        return 0
    lax.fori_loop(0, _YPW // 16, _init, 0)

    def _zero(i, _):
        zeros_v[pl.ds(i * 16, 16)] = jnp.zeros((16,), jnp.int32)
        return 0
    lax.fori_loop(0, 1024 // 16, _zero, 0)

    @pl.when(sid == 0)
    def _():
        pltpu.sync_copy(zeros_v, hist_sh)
    plsc.subcore_barrier()
    pltpu.sync_copy(ones_v, hist_sh.at[yh_v], add=True)
    plsc.subcore_barrier()

    @pl.when(sid == 0)
    def _():
        pltpu.sync_copy(hist_sh, hist_hbm.at[pl.ds(cid * 1024, 1024)])


_sc_hist = functools.partial(
    pl.kernel,
    out_type=[jax.ShapeDtypeStruct((2 * 1024,), jnp.int32)],
    mesh=plsc.VectorSubcoreMesh(core_axis_name="c", subcore_axis_name="s"),
    scratch_types=[
        pltpu.VMEM((_YPW,), jnp.int32),
        pltpu.VMEM((_YPW,), jnp.int32),
        pltpu.VMEM((1024,), jnp.int32),
        pltpu.VMEM_SHARED((1024,), jnp.int32),
    ],
)(_sc_hist_body)


def _tc1_body(out_ref, y_ref, acc_ref):
    i = pl.program_id(0)
    C, blk = out_ref.shape

    @pl.when(i == 0)
    def _init():
        acc_ref[0, 0] = jnp.float32(0.0)

    x = out_ref[...]
    yv = y_ref[0, ...]
    row = lax.broadcasted_iota(jnp.int32, (C, blk), 0)
    onehot = row == yv
    s = jnp.sum(x, axis=0, keepdims=True)
    picked = jnp.sum(jnp.where(onehot, x, 0.0), axis=0, keepdims=True)
    acc_ref[0, 0] += jnp.sum(jnp.log(s) - jnp.log(picked))


def _combine_body(hist_ref, acc_ref, loss_ref):
    hist = hist_ref[...].reshape(2, 8, 128).astype(jnp.float32)
    cnt = jnp.sum(hist, axis=0)
    cterm = jnp.sum(cnt * jnp.log(jnp.maximum(cnt, 1.0)))
    nf = jnp.float32(_N)
    loss_ref[0, 0] = jnp.log(nf) + (acc_ref[0, 0] - cterm) / nf


@jax.jit
def _arb_loss(output, y):
    y = y.astype(jnp.int32)
    (hist,) = _sc_hist(y)
    acc = pl.pallas_call(
        _tc1_body,
        grid=(_N // _BLK,),
        in_specs=[
            pl.BlockSpec((_C, _BLK), lambda i: (0, i)),
            pl.BlockSpec((1, 1, _BLK), lambda i: (i, 0, 0)),
        ],
        out_specs=pl.BlockSpec(memory_space=pltpu.SMEM),
        out_shape=jax.ShapeDtypeStruct((1, 1), jnp.float32),
        compiler_params=pltpu.CompilerParams(
            dimension_semantics=("arbitrary",),
        ),
    )(output.T, y.reshape(_N // _BLK, 1, _BLK))
    out = pl.pallas_call(
        _combine_body,
        out_specs=pl.BlockSpec(memory_space=pltpu.SMEM),
        out_shape=jax.ShapeDtypeStruct((1, 1), jnp.float32),
        in_specs=[
            pl.BlockSpec((16, 128), lambda: (0, 0)),
            pl.BlockSpec(memory_space=pltpu.SMEM),
        ],
    )(hist.reshape(16, 128), acc)
    return out.reshape(())


def kernel(output, y):
    return _arb_loss(output, y)


# speedup vs baseline: 1.0886x; 1.0886x over previous
import functools

import jax
import jax.numpy as jnp
from jax import lax
from jax.experimental import pallas as pl
from jax.experimental.pallas import tpu as pltpu
from jax.experimental.pallas import tpu_sc as plsc

_N = 16384
_C = 1000
_BLK = 2048
_NW = 32
_YPW = _N // _NW


def _sc_hist_body(y_hbm, hist_hbm, yh_v, ones_v, zeros_v, hist_sh):
    cid = lax.axis_index("c")
    sid = lax.axis_index("s")
    wid = sid * 2 + cid

    ones16i = jnp.ones((16,), jnp.int32)
    pltpu.sync_copy(y_hbm.at[pl.ds(wid * _YPW, _YPW)], yh_v)

    def _init(i, _):
        ones_v[pl.ds(i * 16, 16)] = ones16i
        return 0
    lax.fori_loop(0, _YPW // 16, _init, 0)

    def _zero(i, _):
        zeros_v[pl.ds(i * 16, 16)] = jnp.zeros((16,), jnp.int32)
        return 0
    lax.fori_loop(0, 1024 // 16, _zero, 0)

    @pl.when(sid == 0)
    def _():
        pltpu.sync_copy(zeros_v, hist_sh)
    plsc.subcore_barrier()
    pltpu.sync_copy(ones_v, hist_sh.at[yh_v], add=True)
    plsc.subcore_barrier()

    @pl.when(sid == 0)
    def _():
        pltpu.sync_copy(hist_sh, hist_hbm.at[pl.ds(cid * 1024, 1024)])


_sc_hist = functools.partial(
    pl.kernel,
    out_type=[jax.ShapeDtypeStruct((2 * 1024,), jnp.int32)],
    mesh=plsc.VectorSubcoreMesh(core_axis_name="c", subcore_axis_name="s"),
    scratch_types=[
        pltpu.VMEM((_YPW,), jnp.int32),
        pltpu.VMEM((_YPW,), jnp.int32),
        pltpu.VMEM((1024,), jnp.int32),
        pltpu.VMEM_SHARED((1024,), jnp.int32),
    ],
)(_sc_hist_body)


def _tc1_body(out_ref, y_ref, acc_ref):
    i = pl.program_id(0)
    C, blk = out_ref.shape

    @pl.when(i == 0)
    def _init():
        acc_ref[0, 0] = jnp.float32(0.0)

    x = out_ref[...]
    yv = y_ref[0, ...]
    row = lax.broadcasted_iota(jnp.int32, (C, blk), 0)
    onehot = row == yv
    s = jnp.sum(x, axis=0, keepdims=True)
    picked = jnp.sum(jnp.where(onehot, x, 0.0), axis=0, keepdims=True)
    acc_ref[0, 0] += jnp.sum(jnp.log(s) - jnp.log(picked))


def _combine_body(hist_ref, acc_ref, loss_ref):
    hist = hist_ref[...].reshape(2, 8, 128).astype(jnp.float32)
    cnt = jnp.sum(hist, axis=0)
    cterm = jnp.sum(cnt * jnp.log(jnp.maximum(cnt, 1.0)))
    nf = jnp.float32(_N)
    loss_ref[0, 0] = jnp.log(nf) + (acc_ref[0, 0] - cterm) / nf


@jax.jit
def _arb_loss(output, y):
    y = y.astype(jnp.int32)
    (hist,) = _sc_hist(y)
    acc = pl.pallas_call(
        _tc1_body,
        grid=(_N // _BLK,),
        in_specs=[
            pl.BlockSpec((_C, _BLK), lambda i: (0, i)),
            pl.BlockSpec((1, 1, _BLK), lambda i: (i, 0, 0)),
        ],
        out_specs=pl.BlockSpec(memory_space=pltpu.SMEM),
        out_shape=jax.ShapeDtypeStruct((1, 1), jnp.float32),
        compiler_params=pltpu.CompilerParams(
            dimension_semantics=("arbitrary",),
        ),
    )(output.T, y.reshape(_N // _BLK, 1, _BLK))
    out = pl.pallas_call(
        _combine_body,
        out_specs=pl.BlockSpec(memory_space=pltpu.SMEM),
        out_shape=jax.ShapeDtypeStruct((1, 1), jnp.float32),
        in_specs=[
            pl.BlockSpec((16, 128), lambda: (0, 0)),
            pl.BlockSpec(memory_space=pltpu.SMEM),
        ],
    )(hist.reshape(16, 128), acc)
    return out.reshape(())


def kernel(output, y):
    return _arb_loss(output, y)
